# baseline (speedup 1.0000x reference)
import jax
import jax.numpy as jnp
from jax import lax
from jax.experimental import pallas as pl
from jax.experimental.pallas import tpu as pltpu

_HI = lax.Precision.HIGHEST

_B = 200
_V = 48
_H = 64
_NUM_LAYERS = 2
_NUM_SW = 10
_M_EDGES = (_V - 1) + _NUM_SW
_NB = 4
_EU = 64
_ES = 16


def _upper_onehots(t_upper, t_lower, cm, n_slots, f32, transposed=False):
    V = t_upper.shape[0]
    c = jnp.dot(t_upper, cm['right_incl'], preferred_element_type=f32,
                precision=_HI)
    rtot_col = c[:, V - 1:V]
    roff_col = jnp.dot(cm['below_excl'], rtot_col,
                       preferred_element_type=f32, precision=_HI)
    rtot_row = jnp.sum(t_lower, axis=0, keepdims=True)
    roff_row = jnp.dot(rtot_row, cm['strict_upper'],
                       preferred_element_type=f32, precision=_HI)
    k_col = lax.broadcasted_iota(jnp.int32, (n_slots, 1), 0).astype(f32) + 1.0
    row_oh = ((k_col > roff_row) & (k_col <= roff_row + rtot_row)).astype(f32)
    rank = k_col - jnp.dot(row_oh, roff_col, preferred_element_type=f32,
                           precision=_HI)
    csel = jnp.dot(row_oh, c, preferred_element_type=f32, precision=_HI)
    tsel = jnp.dot(row_oh, t_upper, preferred_element_type=f32, precision=_HI)
    col_oh = jnp.where(jnp.abs(csel - rank) < 0.5, tsel, 0.0)
    if not transposed:
        return row_oh, col_oh
    k_row = lax.broadcasted_iota(jnp.int32, (1, n_slots), 1).astype(f32) + 1.0
    row_ohT = ((k_row > roff_col) & (k_row <= roff_col + rtot_col)).astype(f32)
    cT = jnp.dot(cm['left_incl'], t_lower, preferred_element_type=f32,
                 precision=_HI)
    rankT = k_row - jnp.dot(roff_row, row_ohT, preferred_element_type=f32,
                            precision=_HI)
    cselT = jnp.dot(cT, row_ohT, preferred_element_type=f32, precision=_HI)
    tselT = jnp.dot(t_lower, row_ohT, preferred_element_type=f32,
                    precision=_HI)
    col_ohT = jnp.where(jnp.abs(cselT - rankT) < 0.5, tselT, 0.0)
    i_row = jnp.dot(cm['iota_row'], row_ohT, preferred_element_type=f32,
                    precision=_HI)
    j_row = jnp.dot(cm['iota_row'], col_ohT, preferred_element_type=f32,
                    precision=_HI)
    fp_row = i_row * V + j_row
    return row_oh, col_oh, row_ohT, col_ohT, fp_row


def _fwd_kernel(x_ref, a_ref, s_ref, emb_ref, wuv_ref, wge_ref, we12_ref,
                sW1_ref, sb1_ref, sW2_ref, sb2_ref,
                cW1_ref, cb1_ref, cW2_ref, cb2_ref, out_ref):
    NB, V, H = _NB, _V, _H
    f32 = jnp.float32

    A3 = a_ref[...]
    S3 = s_ref[...]
    X3 = x_ref[...]

    mask3 = ((A3 + S3) > 0).astype(f32)
    inv_deg3 = 1.0 / (jnp.sum(mask3, axis=2, keepdims=True) + 1e-6)

    e0 = emb_ref[0:1, :]
    e1 = emb_ref[1:2, :]

    ii = lax.broadcasted_iota(jnp.int32, (V, V), 0)
    jj = lax.broadcasted_iota(jnp.int32, (V, V), 1)
    triu = (jj > ii).astype(f32)
    tril = (ii > jj).astype(f32)
    cm = {
        'right_incl': (ii <= jj).astype(f32),
        'below_excl': (jj < ii).astype(f32),
        'left_incl': (ii >= jj).astype(f32),
        'strict_upper': triu,
        'iota_row': lax.broadcasted_iota(jnp.int32, (1, V), 1).astype(f32),
    }
    iota_col = lax.broadcasted_iota(jnp.int32, (V, 1), 0).astype(f32)

    for nb in range(NB):
        mask = mask3[nb]
        S_nb = S3[nb]
        A_nb = A3[nb]
        x_nb = X3[nb]
        inv_deg = inv_deg3[nb]

        tU = mask * triu
        tL = mask * tril
        rowU, colU, rowUT, colUT, fp_row = _upper_onehots(
            tU, tL, cm, _EU, f32, transposed=True)
        row_oh = jnp.concatenate([rowU, colU], axis=0)
        col_oh = jnp.concatenate([colU, rowU], axis=0)
        row_ohT = jnp.concatenate([rowUT, colUT], axis=1)

        Ssel = jnp.dot(row_oh, S_nb, preferred_element_type=f32,
                       precision=_HI)
        Sval = jnp.sum(Ssel * col_oh, axis=1, keepdims=True)
        s_e = e0 + Sval * (e1 - e0)

        for l in range(_NUM_LAYERS):
            uv = jnp.dot(x_nb, wuv_ref[l], preferred_element_type=f32)
            Ux = uv[:, :H]
            Vx = uv[:, H:]
            ge = jnp.dot(s_e, wge_ref[l], preferred_element_type=f32)
            gates = jax.nn.sigmoid(ge[:, :H])
            sE0 = ge[:, H:]
            Vxc = jnp.dot(col_oh, Vx, preferred_element_type=f32,
                          precision=_HI)
            contrib = gates * Vxc
            agg = jnp.dot(row_ohT, contrib, preferred_element_type=f32,
                          precision=_HI)
            x_nb = jnp.maximum(Ux + agg * inv_deg, 0.0)
            e12 = jnp.dot(x_nb, we12_ref[l], preferred_element_type=f32)
            xE1e = jnp.dot(row_oh, e12[:, :H], preferred_element_type=f32,
                           precision=_HI)
            xE2e = jnp.dot(col_oh, e12[:, H:], preferred_element_type=f32,
                           precision=_HI)
            s_e = jnp.maximum(sE0 + xE1e + xE2e, 0.0)

        xg = jnp.sum(x_nb, axis=0, keepdims=True)

        tS = S_nb * triu
        tSL = S_nb * tril
        rowS, colS = _upper_onehots(tS, tSL, cm, _ES, f32)
        i_k = jnp.dot(rowS, iota_col, preferred_element_type=f32,
                      precision=_HI)
        j_k = jnp.dot(colS, iota_col, preferred_element_type=f32,
                      precision=_HI)
        idx = i_k * V + j_k
        ohSW = (jnp.abs(idx - fp_row) < 0.5).astype(f32)
        sw = jnp.dot(ohSW, s_e[:_EU], preferred_element_type=f32,
                     precision=_HI)
        x1 = jnp.dot(rowS, x_nb, preferred_element_type=f32, precision=_HI)
        x2 = jnp.dot(colS, x_nb, preferred_element_type=f32, precision=_HI)
        smlp_in = jnp.concatenate(
            [sw, x1, x2, jnp.broadcast_to(xg, (_ES, H))], axis=1)
        hs = jnp.maximum(
            jnp.dot(smlp_in, sW1_ref[...], preferred_element_type=f32)
            + sb1_ref[...], 0.0)
        s_out = jnp.dot(hs, sW2_ref[...],
                        preferred_element_type=f32) + sb2_ref[...]

        tA = A_nb * triu
        tAL = A_nb * tril
        rowA, colA = _upper_onehots(tA, tAL, cm, V, f32)
        xb = jnp.dot(rowA, x_nb, preferred_element_type=f32, precision=_HI)
        xe = jnp.dot(colA, x_nb, preferred_element_type=f32, precision=_HI)
        cmlp_in = jnp.concatenate(
            [xb, xe, jnp.broadcast_to(xg, (V, H))], axis=1)
        hc = jnp.maximum(
            jnp.dot(cmlp_in, cW1_ref[...], preferred_element_type=f32)
            + cb1_ref[...], 0.0)
        c_out = jnp.dot(hc, cW2_ref[...],
                        preferred_element_type=f32) + cb2_ref[...]

        nsw = _NUM_SW
        nbr = _V - 1
        zeros47 = jnp.zeros((nbr, 1), f32)
        col = jnp.concatenate([
            c_out[:nbr, 0:1], s_out[:nsw, 1:2],
            zeros47, jax.nn.sigmoid(s_out[:nsw, 0:1]),
            c_out[:nbr, 1:2], s_out[:nsw, 2:3],
            c_out[:nbr, 2:3], s_out[:nsw, 3:4],
        ], axis=0)
        out_ref[0, :, nb:nb + 1] = col


@jax.jit
def kernel(x, A, S, params):
    f32 = jnp.float32
    H = _H
    lp = params['layers']
    wuv = jnp.stack([jnp.concatenate([l['U'], l['Vm']], axis=1) for l in lp])
    wge = jnp.stack([jnp.concatenate([l['G'], l['E0']], axis=1) for l in lp])
    we12 = jnp.stack([jnp.concatenate([l['E1'], l['E2']], axis=1) for l in lp])
    emb = params['embed']
    sW1 = params['smlp_W1']
    sb1 = params['smlp_b1'].reshape(1, 4 * H)
    sW2 = jnp.zeros((4 * H, 8), f32).at[:, :4].set(params['smlp_W2'])
    sb2 = jnp.zeros((1, 8), f32).at[0, :4].set(params['smlp_b2'])
    cW1 = params['cmlp_W1']
    cb1 = params['cmlp_b1'].reshape(1, 3 * H)
    cW2 = jnp.zeros((3 * H, 8), f32).at[:, :3].set(params['cmlp_W2'])
    cb2 = jnp.zeros((1, 8), f32).at[0, :3].set(params['cmlp_b2'])

    grid = (_B // _NB,)
    full = lambda shape: pl.BlockSpec(shape, lambda i: (0,) * len(shape))
    out = pl.pallas_call(
        _fwd_kernel,
        grid=grid,
        in_specs=[
            pl.BlockSpec((_NB, _V, _H), lambda i: (i, 0, 0)),
            pl.BlockSpec((_NB, _V, _V), lambda i: (i, 0, 0)),
            pl.BlockSpec((_NB, _V, _V), lambda i: (i, 0, 0)),
            full((2, H)),
            full((_NUM_LAYERS, H, 2 * H)),
            full((_NUM_LAYERS, H, 2 * H)),
            full((_NUM_LAYERS, H, 2 * H)),
            full((4 * H, 4 * H)),
            full((1, 4 * H)),
            full((4 * H, 8)),
            full((1, 8)),
            full((3 * H, 3 * H)),
            full((1, 3 * H)),
            full((3 * H, 8)),
            full((1, 8)),
        ],
        out_specs=pl.BlockSpec((1, 4 * _M_EDGES, _NB), lambda i: (i, 0, 0)),
        out_shape=jax.ShapeDtypeStruct((_B // _NB, 4 * _M_EDGES, _NB), f32),
        compiler_params=pltpu.CompilerParams(
            dimension_semantics=("parallel",)),
    )(x, A, S, emb, wuv, wge, we12, sW1, sb1, sW2, sb2, cW1, cb1, cW2, cb2)
    return out.transpose(0, 2, 1).reshape(_B, 4 * _M_EDGES)
